# baseline (speedup 1.0000x reference)
import functools

import jax
import jax.numpy as jnp
import numpy as np
from jax.experimental import pallas as pl
from jax.experimental.pallas import tpu as pltpu
from jax.sharding import Mesh, PartitionSpec as P

try:
    from jax import shard_map as _shard_map_fn
except ImportError:
    from jax.experimental.shard_map import shard_map as _shard_map_fn

_BLOCK = 4096


def _plda_block(s_ref, x_ref, u_ref, y_ref, z_ref):
    x = x_ref[...]
    norm = jnp.sqrt(jnp.sum(x * x, axis=1, keepdims=True))
    norm = jnp.maximum(norm, 1e-12)
    y = (s_ref[0] / norm) * x
    y_ref[...] = y
    z_ref[...] = jnp.dot(y, u_ref[...], preferred_element_type=jnp.float32)


def _plda_pallas(x, scale, Ulda):
    batch, dim = x.shape
    block = min(_BLOCK, batch)
    grid = (batch // block,)
    return pl.pallas_call(
        _plda_block,
        grid=grid,
        in_specs=[
            pl.BlockSpec(memory_space=pltpu.SMEM),
            pl.BlockSpec((block, dim), lambda i: (i, 0)),
            pl.BlockSpec((dim, dim), lambda i: (0, 0)),
        ],
        out_specs=[
            pl.BlockSpec((block, dim), lambda i: (i, 0)),
            pl.BlockSpec((block, dim), lambda i: (i, 0)),
        ],
        out_shape=[
            jax.ShapeDtypeStruct((batch, dim), jnp.float32),
            jax.ShapeDtypeStruct((batch, dim), jnp.float32),
        ],
        compiler_params=pltpu.CompilerParams(
            dimension_semantics=("arbitrary",),
        ),
    )(scale, x, Ulda)


def kernel(x, norm_scale, Ulda):
    batch, _ = x.shape
    scale = jnp.reshape(norm_scale.astype(jnp.float32), (1,))
    devices = jax.devices()
    ndev = len(devices)
    while ndev > 1 and batch % ndev != 0:
        ndev -= 1
    if ndev == 1:
        y, z = _plda_pallas(x, scale, Ulda)
        return (y, z)
    mesh = Mesh(np.array(devices[:ndev]), ("b",))
    sharded = _shard_map_fn(
        _plda_pallas,
        mesh=mesh,
        in_specs=(P("b", None), P(None), P(None, None)),
        out_specs=(P("b", None), P("b", None)),
        check_vma=False,
    )
    y, z = sharded(x, scale, Ulda)
    return (y, z)


# speedup vs baseline: 26.5688x; 26.5688x over previous
import jax
import jax.numpy as jnp
from jax.experimental import pallas as pl
from jax.experimental.pallas import tpu as pltpu

_BLOCK = 4096


def _plda_block(s_ref, x_ref, u_ref, y_ref, z_ref):
    x = x_ref[...]
    norm = jnp.sqrt(jnp.sum(x * x, axis=1, keepdims=True))
    norm = jnp.maximum(norm, 1e-12)
    y = (s_ref[0] / norm) * x
    y_ref[...] = y
    z_ref[...] = jnp.dot(y, u_ref[...], preferred_element_type=jnp.float32)


def kernel(x, norm_scale, Ulda):
    batch, dim = x.shape
    scale = jnp.reshape(norm_scale.astype(jnp.float32), (1,))
    block = min(_BLOCK, batch)
    grid = (batch // block,)
    y, z = pl.pallas_call(
        _plda_block,
        grid=grid,
        in_specs=[
            pl.BlockSpec(memory_space=pltpu.SMEM),
            pl.BlockSpec((block, dim), lambda i: (i, 0)),
            pl.BlockSpec((dim, dim), lambda i: (0, 0)),
        ],
        out_specs=[
            pl.BlockSpec((block, dim), lambda i: (i, 0)),
            pl.BlockSpec((block, dim), lambda i: (i, 0)),
        ],
        out_shape=[
            jax.ShapeDtypeStruct((batch, dim), jnp.float32),
            jax.ShapeDtypeStruct((batch, dim), jnp.float32),
        ],
        compiler_params=pltpu.CompilerParams(
            dimension_semantics=("parallel",),
        ),
    )(scale, x, Ulda)
    return (y, z)
